# baseline (speedup 1.0000x reference)
import functools

import jax
import jax.numpy as jnp
from jax.experimental import pallas as pl
from jax.experimental.pallas import tpu as pltpu


def _rank_body(codes_ref, order_ref, inv_ref):
    K, M = codes_ref.shape
    iota_i = jax.lax.broadcasted_iota(jnp.int32, (M, M), 0)
    iota_j = jax.lax.broadcasted_iota(jnp.int32, (M, M), 1)
    for r in range(K):
        c = codes_ref[r, :]
        ci = jnp.reshape(c, (M, 1))
        cj = jnp.reshape(c, (1, M))
        before = (cj < ci) | ((cj == ci) & (iota_j < iota_i))
        rank = jnp.sum(before.astype(jnp.int32), axis=1, keepdims=True)
        inv_ref[r, :] = jnp.reshape(rank, (M,))
        onehot = (rank == iota_j)
        order = jnp.sum(jnp.where(onehot, iota_i, 0), axis=0)
        order_ref[r, :] = order


def _argsort_inverse(codes):
    K, M = codes.shape
    return pl.pallas_call(
        _rank_body,
        out_shape=[
            jax.ShapeDtypeStruct((K, M), jnp.int32),
            jax.ShapeDtypeStruct((K, M), jnp.int32),
        ],
    )(codes)


_SUB = 128
_LANE = 128
_BIG = 1 << 30


def _fps_body(cxyz_ref, sel_ref, sxyz_ref):
    B = cxyz_ref.shape[0]
    m_rows = sel_ref.shape[1]
    iota_n = (jax.lax.broadcasted_iota(jnp.int32, (_SUB, _LANE), 0) * _LANE
              + jax.lax.broadcasted_iota(jnp.int32, (_SUB, _LANE), 1))
    iota_m = (jax.lax.broadcasted_iota(jnp.int32, (m_rows, _LANE), 0) * _LANE
              + jax.lax.broadcasted_iota(jnp.int32, (m_rows, _LANE), 1))
    lane_iota = jax.lax.broadcasted_iota(jnp.int32, (1, _LANE), 1)
    num_sel = m_rows * _LANE

    comps = [[cxyz_ref[b, c] for c in range(3)] for b in range(B)]
    inits = []
    for b in range(B):
        cx, cy, cz = comps[b]
        mask0 = iota_n == 0
        sx = jnp.sum(jnp.where(mask0, cx, 0.0))
        sy = jnp.sum(jnp.where(mask0, cy, 0.0))
        sz = jnp.sum(jnp.where(mask0, cz, 0.0))
        dist = jnp.full((_SUB, _LANE), jnp.inf, jnp.float32)
        sel = jnp.zeros((m_rows, _LANE), jnp.int32)
        at0 = iota_m == 0
        selx = jnp.where(at0, sx, 0.0)
        sely = jnp.where(at0, sy, 0.0)
        selz = jnp.where(at0, sz, 0.0)
        inits.append((dist, sx, sy, sz, sel, selx, sely, selz))

    def body(i, st):
        new = []
        for b in range(B):
            dist, sx, sy, sz, sel, selx, sely, selz = st[b]
            cx, cy, cz = comps[b]
            dx = cx - sx
            dy = cy - sy
            dz = cz - sz
            d = (dx * dx + dy * dy) + dz * dz
            dist = jnp.minimum(dist, d)
            m = jnp.max(dist)
            idx = jnp.min(jnp.where(dist == m, iota_n, _BIG))
            row = idx // _LANE
            col = idx % _LANE
            lsel = lane_iota == col
            nx = jnp.sum(jnp.where(lsel, cxyz_ref[b, 0, pl.ds(row, 1), :], 0.0))
            ny = jnp.sum(jnp.where(lsel, cxyz_ref[b, 1, pl.ds(row, 1), :], 0.0))
            nz = jnp.sum(jnp.where(lsel, cxyz_ref[b, 2, pl.ds(row, 1), :], 0.0))
            at_i = iota_m == i
            sel = jnp.where(at_i, idx, sel)
            selx = jnp.where(at_i, nx, selx)
            sely = jnp.where(at_i, ny, sely)
            selz = jnp.where(at_i, nz, selz)
            new.append((dist, nx, ny, nz, sel, selx, sely, selz))
        return tuple(new)

    final = jax.lax.fori_loop(1, num_sel, body, tuple(inits))
    for b in range(B):
        _, _, _, _, sel, selx, sely, selz = final[b]
        sel_ref[b] = sel
        sxyz_ref[b, 0] = selx
        sxyz_ref[b, 1] = sely
        sxyz_ref[b, 2] = selz


_KNN_R = 128
_CHUNK = 128
_NCAND = 36


def _knn_p1_body(c_ref, sxyz_ref, d_ref, cid_ref):
    N = c_ref.shape[2]
    R = sxyz_ref.shape[2]
    T = cid_ref.shape[2]
    nchunk = N // _CHUNK
    comps = []
    for c in range(3):
        cc = jnp.reshape(c_ref[0, c, :], (1, N))
        sc = jnp.reshape(sxyz_ref[0, c, :], (R, 1))
        comps.append((sc, cc))
    dx = comps[0][0] - comps[0][1]
    dy = comps[1][0] - comps[1][1]
    dz = comps[2][0] - comps[2][1]
    d = (dx * dx + dy * dy) + dz * dz
    d_ref[0] = d
    cmin = jnp.min(d.reshape(R, nchunk, _CHUNK), axis=2)
    cgidx = jax.lax.broadcasted_iota(jnp.int32, (R, nchunk), 1)
    mprev = jnp.full((R, 1), -jnp.inf, jnp.float32)
    iprev = jnp.full((R, 1), -1, jnp.int32)
    for t in range(T):
        cond = (cmin > mprev) | ((cmin == mprev) & (cgidx > iprev))
        m = jnp.min(jnp.where(cond, cmin, jnp.inf), axis=1, keepdims=True)
        cid = jnp.min(jnp.where(cond & (cmin == m), cgidx, _BIG),
                      axis=1, keepdims=True)
        cid_ref[0, :, t : t + 1] = cid
        mprev, iprev = m, cid


def _knn_p2_body(cand_ref, gidx_ref, nidx_ref):
    M = cand_ref.shape[1]
    G = nidx_ref.shape[2]
    vals = cand_ref[0]
    g = gidx_ref[0]
    mprev = jnp.full((M, 1), -jnp.inf, jnp.float32)
    iprev = jnp.full((M, 1), -1, jnp.int32)
    for k in range(G):
        cond = (vals > mprev) | ((vals == mprev) & (g > iprev))
        m = jnp.min(jnp.where(cond, vals, jnp.inf), axis=1, keepdims=True)
        idx = jnp.min(jnp.where(cond & (vals == m), g, _BIG),
                      axis=1, keepdims=True)
        nidx_ref[0, :, k : k + 1] = idx
        mprev, iprev = m, idx


def _sc_compact(table, idx):
    from jax.experimental.pallas import tpu_sc as plsc

    NI = idx.shape[0]
    NW = 32
    NSUB = 4
    b_per_w = NI // NW
    b_sub = b_per_w // NSUB
    D = table.shape[1]
    mesh = plsc.VectorSubcoreMesh(core_axis_name="c", subcore_axis_name="s")

    @functools.partial(
        pl.kernel,
        mesh=mesh,
        out_type=jax.ShapeDtypeStruct((NI, D), jnp.float32),
        scratch_types=[
            pltpu.VMEM((b_sub,), jnp.int32),
            pltpu.VMEM((b_sub, D), jnp.float32),
            pltpu.SemaphoreType.DMA,
        ],
    )
    def gath(table_hbm, idx_hbm, out_hbm, idx_v, rows_v, sem):
        wid = jax.lax.axis_index("s") * 2 + jax.lax.axis_index("c")
        for g in range(NSUB):
            base = wid * b_per_w + g * b_sub
            pltpu.sync_copy(idx_hbm.at[pl.ds(base, b_sub)], idx_v)
            pltpu.async_copy(table_hbm.at[idx_v], rows_v, sem).wait()
            pltpu.sync_copy(rows_v, out_hbm.at[pl.ds(base, b_sub)])

    return gath(table, idx)


def _knn_pallas(c_t, sxyz_t, B, n_per, num_group, group_size):
    N, M, T = n_per, num_group, _NCAND
    nblk = M // _KNN_R
    nchunk = N // _CHUNK
    d_out, chunkids = pl.pallas_call(
        _knn_p1_body,
        grid=(B, nblk),
        in_specs=[
            pl.BlockSpec((1, 3, N), lambda b, rb: (b, 0, 0)),
            pl.BlockSpec((1, 3, _KNN_R), lambda b, rb: (b, 0, rb)),
        ],
        out_specs=[
            pl.BlockSpec((1, _KNN_R, N), lambda b, rb: (b, rb, 0)),
            pl.BlockSpec((1, _KNN_R, T), lambda b, rb: (b, rb, 0)),
        ],
        out_shape=[
            jax.ShapeDtypeStruct((B, M, N), jnp.float32),
            jax.ShapeDtypeStruct((B, M, T), jnp.int32),
        ],
    )(c_t, sxyz_t)

    row_base = (jnp.arange(B, dtype=jnp.int32)[:, None, None] * M
                + jnp.arange(M, dtype=jnp.int32)[None, :, None]) * nchunk
    idx_flat = (row_base + chunkids).reshape(-1)
    table = d_out.reshape(B * M * nchunk, _CHUNK)
    compact = _sc_compact(table, idx_flat).reshape(B, M, T * _CHUNK)
    gidx = (chunkids[..., None] * _CHUNK
            + jnp.arange(_CHUNK, dtype=jnp.int32)).reshape(B, M, T * _CHUNK)

    return pl.pallas_call(
        _knn_p2_body,
        grid=(B,),
        in_specs=[
            pl.BlockSpec((1, M, T * _CHUNK), lambda b: (b, 0, 0)),
            pl.BlockSpec((1, M, T * _CHUNK), lambda b: (b, 0, 0)),
        ],
        out_specs=pl.BlockSpec((1, M, group_size), lambda b: (b, 0, 0)),
        out_shape=jax.ShapeDtypeStruct((B, M, group_size), jnp.int32),
    )(compact, gidx)


def _fps_pallas(coord, B, n_per, num_group):
    m_rows = num_group // _LANE
    c_t = coord.reshape(B, n_per, 3).transpose(0, 2, 1).reshape(B, 3, _SUB, _LANE)
    sel8, sxyz8 = pl.pallas_call(
        _fps_body,
        out_shape=[
            jax.ShapeDtypeStruct((B, m_rows, _LANE), jnp.int32),
            jax.ShapeDtypeStruct((B, 3, m_rows, _LANE), jnp.float32),
        ],
    )(c_t)
    local_idx = sel8.reshape(B, num_group)
    s_xyz_b = sxyz8.reshape(B, 3, num_group).transpose(0, 2, 1)
    return local_idx, s_xyz_b


def kernel(coord, offset, batch_bin, serialized_code):
    B = offset.shape[0]
    n_total = coord.shape[0]
    n_per = n_total // B
    num_group = 512
    group_size = 32

    local_idx, s_xyz_b = _fps_pallas(coord, B, n_per, num_group)
    base = (jnp.arange(B, dtype=jnp.int32) * n_per)[:, None]
    s_idx = (local_idx + base).reshape(-1)

    c_t = coord.reshape(B, n_per, 3).transpose(0, 2, 1)
    sxyz_t = s_xyz_b.transpose(0, 2, 1)
    nidx_local = _knn_pallas(c_t, sxyz_t, B, n_per, num_group, group_size)
    nidx = (nidx_local + base[:, :, None]).reshape(B * num_group, group_size)

    s_xyz = s_xyz_b.reshape(B * num_group, 3)
    s_n = coord[nidx] - s_xyz[:, None, :]
    s_n = s_n[:, 1:, :]
    codes = serialized_code[:, s_idx]
    s_order, s_inverse = _argsort_inverse(codes)
    return (s_idx, s_n, s_xyz, s_order, s_inverse)


# speedup vs baseline: 3.4229x; 3.4229x over previous
import functools

import jax
import jax.numpy as jnp
from jax.experimental import pallas as pl
from jax.experimental.pallas import tpu as pltpu


def _rank_body(codes_ref, order_ref, inv_ref):
    K, M = codes_ref.shape
    iota_i = jax.lax.broadcasted_iota(jnp.int32, (M, M), 0)
    iota_j = jax.lax.broadcasted_iota(jnp.int32, (M, M), 1)
    for r in range(K):
        c = codes_ref[r, :]
        ci = jnp.reshape(c, (M, 1))
        cj = jnp.reshape(c, (1, M))
        before = (cj < ci) | ((cj == ci) & (iota_j < iota_i))
        rank = jnp.sum(before.astype(jnp.int32), axis=1, keepdims=True)
        inv_ref[r, :] = jnp.reshape(rank, (M,))
        onehot = (rank == iota_j)
        order = jnp.sum(jnp.where(onehot, iota_i, 0), axis=0)
        order_ref[r, :] = order


def _argsort_inverse(codes):
    K, M = codes.shape
    return pl.pallas_call(
        _rank_body,
        out_shape=[
            jax.ShapeDtypeStruct((K, M), jnp.int32),
            jax.ShapeDtypeStruct((K, M), jnp.int32),
        ],
    )(codes)


_SUB = 128
_LANE = 128
_BIG = 1 << 30


def _fps_body(cxyz_ref, sel_ref, sxyz_ref):
    B = cxyz_ref.shape[0]
    m_rows = sel_ref.shape[1]
    iota_n = (jax.lax.broadcasted_iota(jnp.int32, (_SUB, _LANE), 0) * _LANE
              + jax.lax.broadcasted_iota(jnp.int32, (_SUB, _LANE), 1))
    iota_m = (jax.lax.broadcasted_iota(jnp.int32, (m_rows, _LANE), 0) * _LANE
              + jax.lax.broadcasted_iota(jnp.int32, (m_rows, _LANE), 1))
    lane_iota = jax.lax.broadcasted_iota(jnp.int32, (1, _LANE), 1)
    num_sel = m_rows * _LANE

    comps = [[cxyz_ref[b, c] for c in range(3)] for b in range(B)]
    inits = []
    for b in range(B):
        cx, cy, cz = comps[b]
        mask0 = iota_n == 0
        sx = jnp.sum(jnp.where(mask0, cx, 0.0))
        sy = jnp.sum(jnp.where(mask0, cy, 0.0))
        sz = jnp.sum(jnp.where(mask0, cz, 0.0))
        dist = jnp.full((_SUB, _LANE), jnp.inf, jnp.float32)
        sel = jnp.zeros((m_rows, _LANE), jnp.int32)
        at0 = iota_m == 0
        selx = jnp.where(at0, sx, 0.0)
        sely = jnp.where(at0, sy, 0.0)
        selz = jnp.where(at0, sz, 0.0)
        inits.append((dist, sx, sy, sz, sel, selx, sely, selz))

    def body(i, st):
        new = []
        for b in range(B):
            dist, sx, sy, sz, sel, selx, sely, selz = st[b]
            cx, cy, cz = comps[b]
            dx = cx - sx
            dy = cy - sy
            dz = cz - sz
            d = (dx * dx + dy * dy) + dz * dz
            dist = jnp.minimum(dist, d)
            m = jnp.max(dist)
            idx = jnp.min(jnp.where(dist == m, iota_n, _BIG))
            row = idx // _LANE
            col = idx % _LANE
            lsel = lane_iota == col
            nx = jnp.sum(jnp.where(lsel, cxyz_ref[b, 0, pl.ds(row, 1), :], 0.0))
            ny = jnp.sum(jnp.where(lsel, cxyz_ref[b, 1, pl.ds(row, 1), :], 0.0))
            nz = jnp.sum(jnp.where(lsel, cxyz_ref[b, 2, pl.ds(row, 1), :], 0.0))
            at_i = iota_m == i
            sel = jnp.where(at_i, idx, sel)
            selx = jnp.where(at_i, nx, selx)
            sely = jnp.where(at_i, ny, sely)
            selz = jnp.where(at_i, nz, selz)
            new.append((dist, nx, ny, nz, sel, selx, sely, selz))
        return tuple(new)

    final = jax.lax.fori_loop(1, num_sel, body, tuple(inits))
    for b in range(B):
        _, _, _, _, sel, selx, sely, selz = final[b]
        sel_ref[b] = sel
        sxyz_ref[b, 0] = selx
        sxyz_ref[b, 1] = sely
        sxyz_ref[b, 2] = selz


_KNN_R = 128
_CHUNK = 128
_NCAND = 36


def _knn_p1_body(c_ref, sxyz_ref, d_ref, cid_ref):
    N = c_ref.shape[2]
    R = sxyz_ref.shape[2]
    T = cid_ref.shape[2]
    nchunk = N // _CHUNK
    comps = []
    for c in range(3):
        cc = jnp.reshape(c_ref[0, c, :], (1, N))
        sc = jnp.reshape(sxyz_ref[0, c, :], (R, 1))
        comps.append((sc, cc))
    dx = comps[0][0] - comps[0][1]
    dy = comps[1][0] - comps[1][1]
    dz = comps[2][0] - comps[2][1]
    d = (dx * dx + dy * dy) + dz * dz
    d_ref[0] = d
    cmin = jnp.min(d.reshape(R, nchunk, _CHUNK), axis=2)
    cgidx = jax.lax.broadcasted_iota(jnp.int32, (R, nchunk), 1)
    for t in range(T):
        m = jnp.min(cmin, axis=1, keepdims=True)
        cid = jnp.min(jnp.where(cmin == m, cgidx, _BIG), axis=1, keepdims=True)
        cid_ref[0, :, t : t + 1] = cid
        cmin = jnp.where(cgidx == cid, jnp.inf, cmin)


def _knn_p2_body(cand_ref, gidx_ref, nidx_ref, vals_ref):
    G = nidx_ref.shape[2]
    vals_ref[...] = cand_ref[0]
    g = gidx_ref[0]
    for k in range(G):
        vals = vals_ref[...]
        m = jnp.min(vals, axis=1, keepdims=True)
        idx = jnp.min(jnp.where(vals == m, g, _BIG), axis=1, keepdims=True)
        nidx_ref[0, :, k : k + 1] = idx
        vals_ref[...] = jnp.where(g == idx, jnp.inf, vals)


def _sc_compact(table, idx):
    from jax.experimental.pallas import tpu_sc as plsc

    NI = idx.shape[0]
    NW = 32
    NSUB = 4
    b_per_w = NI // NW
    b_sub = b_per_w // NSUB
    D = table.shape[1]
    mesh = plsc.VectorSubcoreMesh(core_axis_name="c", subcore_axis_name="s")

    @functools.partial(
        pl.kernel,
        mesh=mesh,
        out_type=jax.ShapeDtypeStruct((NI, D), jnp.float32),
        scratch_types=[
            pltpu.VMEM((b_sub,), jnp.int32),
            pltpu.VMEM((b_sub, D), jnp.float32),
            pltpu.SemaphoreType.DMA,
        ],
    )
    def gath(table_hbm, idx_hbm, out_hbm, idx_v, rows_v, sem):
        wid = jax.lax.axis_index("s") * 2 + jax.lax.axis_index("c")
        for g in range(NSUB):
            base = wid * b_per_w + g * b_sub
            pltpu.sync_copy(idx_hbm.at[pl.ds(base, b_sub)], idx_v)
            pltpu.async_copy(table_hbm.at[idx_v], rows_v, sem).wait()
            pltpu.sync_copy(rows_v, out_hbm.at[pl.ds(base, b_sub)])

    return gath(table, idx)


def _knn_pallas(c_t, sxyz_t, B, n_per, num_group, group_size):
    N, M, T = n_per, num_group, _NCAND
    nblk = M // _KNN_R
    nchunk = N // _CHUNK
    d_out, chunkids = pl.pallas_call(
        _knn_p1_body,
        grid=(B, nblk),
        in_specs=[
            pl.BlockSpec((1, 3, N), lambda b, rb: (b, 0, 0)),
            pl.BlockSpec((1, 3, _KNN_R), lambda b, rb: (b, 0, rb)),
        ],
        out_specs=[
            pl.BlockSpec((1, _KNN_R, N), lambda b, rb: (b, rb, 0)),
            pl.BlockSpec((1, _KNN_R, T), lambda b, rb: (b, rb, 0)),
        ],
        out_shape=[
            jax.ShapeDtypeStruct((B, M, N), jnp.float32),
            jax.ShapeDtypeStruct((B, M, T), jnp.int32),
        ],
    )(c_t, sxyz_t)

    row_base = (jnp.arange(B, dtype=jnp.int32)[:, None, None] * M
                + jnp.arange(M, dtype=jnp.int32)[None, :, None]) * nchunk
    idx_flat = (row_base + chunkids).reshape(-1)
    table = d_out.reshape(B * M * nchunk, _CHUNK)
    compact = _sc_compact(table, idx_flat).reshape(B, M, T * _CHUNK)
    gidx = (chunkids[..., None] * _CHUNK
            + jnp.arange(_CHUNK, dtype=jnp.int32)).reshape(B, M, T * _CHUNK)

    return pl.pallas_call(
        _knn_p2_body,
        grid=(B,),
        in_specs=[
            pl.BlockSpec((1, M, T * _CHUNK), lambda b: (b, 0, 0)),
            pl.BlockSpec((1, M, T * _CHUNK), lambda b: (b, 0, 0)),
        ],
        out_specs=pl.BlockSpec((1, M, group_size), lambda b: (b, 0, 0)),
        out_shape=jax.ShapeDtypeStruct((B, M, group_size), jnp.int32),
        scratch_shapes=[pltpu.VMEM((M, T * _CHUNK), jnp.float32)],
    )(compact, gidx)


def _fps_pallas(coord, B, n_per, num_group):
    m_rows = num_group // _LANE
    c_t = coord.reshape(B, n_per, 3).transpose(0, 2, 1).reshape(B, 3, _SUB, _LANE)
    sel8, sxyz8 = pl.pallas_call(
        _fps_body,
        out_shape=[
            jax.ShapeDtypeStruct((B, m_rows, _LANE), jnp.int32),
            jax.ShapeDtypeStruct((B, 3, m_rows, _LANE), jnp.float32),
        ],
    )(c_t)
    local_idx = sel8.reshape(B, num_group)
    s_xyz_b = sxyz8.reshape(B, 3, num_group).transpose(0, 2, 1)
    return local_idx, s_xyz_b


def kernel(coord, offset, batch_bin, serialized_code):
    B = offset.shape[0]
    n_total = coord.shape[0]
    n_per = n_total // B
    num_group = 512
    group_size = 32

    local_idx, s_xyz_b = _fps_pallas(coord, B, n_per, num_group)
    base = (jnp.arange(B, dtype=jnp.int32) * n_per)[:, None]
    s_idx = (local_idx + base).reshape(-1)

    c_t = coord.reshape(B, n_per, 3).transpose(0, 2, 1)
    sxyz_t = s_xyz_b.transpose(0, 2, 1)
    nidx_local = _knn_pallas(c_t, sxyz_t, B, n_per, num_group, group_size)
    nidx = (nidx_local + base[:, :, None]).reshape(B * num_group, group_size)

    s_xyz = s_xyz_b.reshape(B * num_group, 3)
    s_n = coord[nidx] - s_xyz[:, None, :]
    s_n = s_n[:, 1:, :]
    codes = serialized_code[:, s_idx]
    s_order, s_inverse = _argsort_inverse(codes)
    return (s_idx, s_n, s_xyz, s_order, s_inverse)


# speedup vs baseline: 3.5121x; 1.0261x over previous
import functools

import jax
import jax.numpy as jnp
from jax.experimental import pallas as pl
from jax.experimental.pallas import tpu as pltpu


def _rank_body(codes_ref, order_ref, inv_ref):
    K, M = codes_ref.shape
    iota_i = jax.lax.broadcasted_iota(jnp.int32, (M, M), 0)
    iota_j = jax.lax.broadcasted_iota(jnp.int32, (M, M), 1)
    for r in range(K):
        c = codes_ref[r, :]
        ci = jnp.reshape(c, (M, 1))
        cj = jnp.reshape(c, (1, M))
        before = (cj < ci) | ((cj == ci) & (iota_j < iota_i))
        rank = jnp.sum(before.astype(jnp.int32), axis=1, keepdims=True)
        inv_ref[r, :] = jnp.reshape(rank, (M,))
        onehot = (rank == iota_j)
        order = jnp.sum(jnp.where(onehot, iota_i, 0), axis=0)
        order_ref[r, :] = order


def _argsort_inverse(codes):
    K, M = codes.shape
    return pl.pallas_call(
        _rank_body,
        out_shape=[
            jax.ShapeDtypeStruct((K, M), jnp.int32),
            jax.ShapeDtypeStruct((K, M), jnp.int32),
        ],
    )(codes)


_SUB = 128
_LANE = 128
_BIG = 1 << 30


def _fps_body(cxyz_ref, sel_ref, sxyz_ref):
    B = cxyz_ref.shape[0]
    m_rows = sel_ref.shape[1]
    iota_n = (jax.lax.broadcasted_iota(jnp.int32, (_SUB, _LANE), 0) * _LANE
              + jax.lax.broadcasted_iota(jnp.int32, (_SUB, _LANE), 1))
    iota_m = (jax.lax.broadcasted_iota(jnp.int32, (m_rows, _LANE), 0) * _LANE
              + jax.lax.broadcasted_iota(jnp.int32, (m_rows, _LANE), 1))
    lane_iota = jax.lax.broadcasted_iota(jnp.int32, (1, _LANE), 1)
    num_sel = m_rows * _LANE

    comps = [[cxyz_ref[b, c] for c in range(3)] for b in range(B)]
    inits = []
    for b in range(B):
        cx, cy, cz = comps[b]
        mask0 = iota_n == 0
        sx = jnp.sum(jnp.where(mask0, cx, 0.0))
        sy = jnp.sum(jnp.where(mask0, cy, 0.0))
        sz = jnp.sum(jnp.where(mask0, cz, 0.0))
        dist = jnp.full((_SUB, _LANE), jnp.inf, jnp.float32)
        sel = jnp.zeros((m_rows, _LANE), jnp.int32)
        at0 = iota_m == 0
        selx = jnp.where(at0, sx, 0.0)
        sely = jnp.where(at0, sy, 0.0)
        selz = jnp.where(at0, sz, 0.0)
        inits.append((dist, sx, sy, sz, sel, selx, sely, selz))

    def body(i, st):
        new = []
        for b in range(B):
            dist, sx, sy, sz, sel, selx, sely, selz = st[b]
            cx, cy, cz = comps[b]
            dx = cx - sx
            dy = cy - sy
            dz = cz - sz
            d = (dx * dx + dy * dy) + dz * dz
            dist = jnp.minimum(dist, d)
            m = jnp.max(dist)
            idx = jnp.min(jnp.where(dist == m, iota_n, _BIG))
            row = idx // _LANE
            col = idx % _LANE
            lsel = lane_iota == col
            nx = jnp.sum(jnp.where(lsel, cxyz_ref[b, 0, pl.ds(row, 1), :], 0.0))
            ny = jnp.sum(jnp.where(lsel, cxyz_ref[b, 1, pl.ds(row, 1), :], 0.0))
            nz = jnp.sum(jnp.where(lsel, cxyz_ref[b, 2, pl.ds(row, 1), :], 0.0))
            at_i = iota_m == i
            sel = jnp.where(at_i, idx, sel)
            selx = jnp.where(at_i, nx, selx)
            sely = jnp.where(at_i, ny, sely)
            selz = jnp.where(at_i, nz, selz)
            new.append((dist, nx, ny, nz, sel, selx, sely, selz))
        return tuple(new)

    final = jax.lax.fori_loop(1, num_sel, body, tuple(inits))
    for b in range(B):
        _, _, _, _, sel, selx, sely, selz = final[b]
        sel_ref[b] = sel
        sxyz_ref[b, 0] = selx
        sxyz_ref[b, 1] = sely
        sxyz_ref[b, 2] = selz


_KNN_R = 128
_CHUNK = 128
_NCAND = 36


def _knn_p1_body(c_ref, sxyz_ref, d_ref, cid_ref, gidx_ref):
    N = c_ref.shape[2]
    R = sxyz_ref.shape[2]
    T = cid_ref.shape[2]
    nchunk = N // _CHUNK
    comps = []
    for c in range(3):
        cc = jnp.reshape(c_ref[0, c, :], (1, N))
        sc = jnp.reshape(sxyz_ref[0, c, :], (R, 1))
        comps.append((sc, cc))
    dx = comps[0][0] - comps[0][1]
    dy = comps[1][0] - comps[1][1]
    dz = comps[2][0] - comps[2][1]
    d = (dx * dx + dy * dy) + dz * dz
    d_ref[0] = d
    cmin = jnp.min(d.reshape(R, nchunk, _CHUNK), axis=2)
    cgidx = jax.lax.broadcasted_iota(jnp.int32, (R, nchunk), 1)
    lane_c = jax.lax.broadcasted_iota(jnp.int32, (1, _CHUNK), 1)
    for t in range(T):
        m = jnp.min(cmin, axis=1, keepdims=True)
        cid = jnp.min(jnp.where(cmin == m, cgidx, _BIG), axis=1, keepdims=True)
        cid_ref[0, :, t : t + 1] = cid
        gidx_ref[0, :, t * _CHUNK : (t + 1) * _CHUNK] = cid * _CHUNK + lane_c
        cmin = jnp.where(cgidx == cid, jnp.inf, cmin)


def _knn_p2_body(cand_ref, gidx_ref, nidx_ref, vals_ref):
    G = nidx_ref.shape[2]
    vals_ref[...] = cand_ref[0]
    g = gidx_ref[0]
    for k in range(G):
        vals = vals_ref[...]
        m = jnp.min(vals, axis=1, keepdims=True)
        idx = jnp.min(jnp.where(vals == m, g, _BIG), axis=1, keepdims=True)
        nidx_ref[0, :, k : k + 1] = idx
        vals_ref[...] = jnp.where(g == idx, jnp.inf, vals)


def _sc_compact(table, idx):
    from jax.experimental.pallas import tpu_sc as plsc

    NI = idx.shape[0]
    NW = 32
    NSUB = 4
    b_per_w = NI // NW
    b_sub = b_per_w // NSUB
    D = table.shape[1]
    mesh = plsc.VectorSubcoreMesh(core_axis_name="c", subcore_axis_name="s")

    @functools.partial(
        pl.kernel,
        mesh=mesh,
        out_type=jax.ShapeDtypeStruct((NI, D), jnp.float32),
        scratch_types=[
            pltpu.VMEM((b_sub,), jnp.int32),
            pltpu.VMEM((b_sub, D), jnp.float32),
            pltpu.SemaphoreType.DMA,
        ],
    )
    def gath(table_hbm, idx_hbm, out_hbm, idx_v, rows_v, sem):
        wid = jax.lax.axis_index("s") * 2 + jax.lax.axis_index("c")
        for g in range(NSUB):
            base = wid * b_per_w + g * b_sub
            pltpu.sync_copy(idx_hbm.at[pl.ds(base, b_sub)], idx_v)
            pltpu.async_copy(table_hbm.at[idx_v], rows_v, sem).wait()
            pltpu.sync_copy(rows_v, out_hbm.at[pl.ds(base, b_sub)])

    return gath(table, idx)


def _knn_pallas(c_t, sxyz_t, B, n_per, num_group, group_size):
    N, M, T = n_per, num_group, _NCAND
    nblk = M // _KNN_R
    nchunk = N // _CHUNK
    d_out, chunkids, gidx = pl.pallas_call(
        _knn_p1_body,
        grid=(B, nblk),
        in_specs=[
            pl.BlockSpec((1, 3, N), lambda b, rb: (b, 0, 0)),
            pl.BlockSpec((1, 3, _KNN_R), lambda b, rb: (b, 0, rb)),
        ],
        out_specs=[
            pl.BlockSpec((1, _KNN_R, N), lambda b, rb: (b, rb, 0)),
            pl.BlockSpec((1, _KNN_R, T), lambda b, rb: (b, rb, 0)),
            pl.BlockSpec((1, _KNN_R, T * _CHUNK), lambda b, rb: (b, rb, 0)),
        ],
        out_shape=[
            jax.ShapeDtypeStruct((B, M, N), jnp.float32),
            jax.ShapeDtypeStruct((B, M, T), jnp.int32),
            jax.ShapeDtypeStruct((B, M, T * _CHUNK), jnp.int32),
        ],
    )(c_t, sxyz_t)

    row_base = (jnp.arange(B, dtype=jnp.int32)[:, None, None] * M
                + jnp.arange(M, dtype=jnp.int32)[None, :, None]) * nchunk
    idx_flat = (row_base + chunkids).reshape(-1)
    table = d_out.reshape(B * M * nchunk, _CHUNK)
    compact = _sc_compact(table, idx_flat).reshape(B, M, T * _CHUNK)

    return pl.pallas_call(
        _knn_p2_body,
        grid=(B,),
        in_specs=[
            pl.BlockSpec((1, M, T * _CHUNK), lambda b: (b, 0, 0)),
            pl.BlockSpec((1, M, T * _CHUNK), lambda b: (b, 0, 0)),
        ],
        out_specs=pl.BlockSpec((1, M, group_size), lambda b: (b, 0, 0)),
        out_shape=jax.ShapeDtypeStruct((B, M, group_size), jnp.int32),
        scratch_shapes=[pltpu.VMEM((M, T * _CHUNK), jnp.float32)],
    )(compact, gidx)


def _fps_pallas(coord, B, n_per, num_group):
    m_rows = num_group // _LANE
    c_t = coord.reshape(B, n_per, 3).transpose(0, 2, 1).reshape(B, 3, _SUB, _LANE)
    sel8, sxyz8 = pl.pallas_call(
        _fps_body,
        out_shape=[
            jax.ShapeDtypeStruct((B, m_rows, _LANE), jnp.int32),
            jax.ShapeDtypeStruct((B, 3, m_rows, _LANE), jnp.float32),
        ],
    )(c_t)
    local_idx = sel8.reshape(B, num_group)
    s_xyz_b = sxyz8.reshape(B, 3, num_group).transpose(0, 2, 1)
    return local_idx, s_xyz_b


def kernel(coord, offset, batch_bin, serialized_code):
    B = offset.shape[0]
    n_total = coord.shape[0]
    n_per = n_total // B
    num_group = 512
    group_size = 32

    local_idx, s_xyz_b = _fps_pallas(coord, B, n_per, num_group)
    base = (jnp.arange(B, dtype=jnp.int32) * n_per)[:, None]
    s_idx = (local_idx + base).reshape(-1)

    c_t = coord.reshape(B, n_per, 3).transpose(0, 2, 1)
    sxyz_t = s_xyz_b.transpose(0, 2, 1)
    nidx_local = _knn_pallas(c_t, sxyz_t, B, n_per, num_group, group_size)
    nidx = (nidx_local + base[:, :, None]).reshape(B * num_group, group_size)

    s_xyz = s_xyz_b.reshape(B * num_group, 3)
    s_n = coord[nidx] - s_xyz[:, None, :]
    s_n = s_n[:, 1:, :]
    codes = serialized_code[:, s_idx]
    s_order, s_inverse = _argsort_inverse(codes)
    return (s_idx, s_n, s_xyz, s_order, s_inverse)


# speedup vs baseline: 3.5168x; 1.0013x over previous
import functools

import jax
import jax.numpy as jnp
from jax.experimental import pallas as pl
from jax.experimental.pallas import tpu as pltpu


def _rank_body(codes_ref, order_ref, inv_ref):
    K, M = codes_ref.shape
    iota_i = jax.lax.broadcasted_iota(jnp.int32, (M, M), 0)
    iota_j = jax.lax.broadcasted_iota(jnp.int32, (M, M), 1)
    for r in range(K):
        c = codes_ref[r, :]
        ci = jnp.reshape(c, (M, 1))
        cj = jnp.reshape(c, (1, M))
        before = (cj < ci) | ((cj == ci) & (iota_j < iota_i))
        rank = jnp.sum(before.astype(jnp.int32), axis=1, keepdims=True)
        inv_ref[r, :] = jnp.reshape(rank, (M,))
        onehot = (rank == iota_j)
        order = jnp.sum(jnp.where(onehot, iota_i, 0), axis=0)
        order_ref[r, :] = order


def _argsort_inverse(codes):
    K, M = codes.shape
    return pl.pallas_call(
        _rank_body,
        out_shape=[
            jax.ShapeDtypeStruct((K, M), jnp.int32),
            jax.ShapeDtypeStruct((K, M), jnp.int32),
        ],
    )(codes)


_SUB = 128
_LANE = 128
_BIG = 1 << 30


def _fps_body(cxyz_ref, sel_ref, sxyz_ref):
    B = cxyz_ref.shape[0]
    m_rows = sel_ref.shape[1]
    iota_n = (jax.lax.broadcasted_iota(jnp.int32, (_SUB, _LANE), 0) * _LANE
              + jax.lax.broadcasted_iota(jnp.int32, (_SUB, _LANE), 1))
    iota_m = (jax.lax.broadcasted_iota(jnp.int32, (m_rows, _LANE), 0) * _LANE
              + jax.lax.broadcasted_iota(jnp.int32, (m_rows, _LANE), 1))
    lane_iota = jax.lax.broadcasted_iota(jnp.int32, (1, _LANE), 1)
    num_sel = m_rows * _LANE

    comps = [[cxyz_ref[b, c] for c in range(3)] for b in range(B)]
    inits = []
    for b in range(B):
        cx, cy, cz = comps[b]
        mask0 = iota_n == 0
        sx = jnp.sum(jnp.where(mask0, cx, 0.0), axis=(0, 1), keepdims=True)
        sy = jnp.sum(jnp.where(mask0, cy, 0.0), axis=(0, 1), keepdims=True)
        sz = jnp.sum(jnp.where(mask0, cz, 0.0), axis=(0, 1), keepdims=True)
        dist = jnp.full((_SUB, _LANE), jnp.inf, jnp.float32)
        sel = jnp.zeros((m_rows, _LANE), jnp.int32)
        at0 = iota_m == 0
        selx = jnp.where(at0, sx, 0.0)
        sely = jnp.where(at0, sy, 0.0)
        selz = jnp.where(at0, sz, 0.0)
        inits.append((dist, sx, sy, sz, sel, selx, sely, selz))

    def body(i, st):
        new = []
        for b in range(B):
            dist, sx, sy, sz, sel, selx, sely, selz = st[b]
            cx, cy, cz = comps[b]
            dx = cx - sx
            dy = cy - sy
            dz = cz - sz
            d = (dx * dx + dy * dy) + dz * dz
            dist = jnp.minimum(dist, d)
            m = jnp.max(dist, axis=(0, 1), keepdims=True)
            idx = jnp.min(jnp.where(dist == m, iota_n, _BIG),
                          axis=(0, 1), keepdims=True)
            row = idx[0, 0] // _LANE
            col = idx % _LANE
            lsel = lane_iota == col
            nx = jnp.sum(jnp.where(lsel, cxyz_ref[b, 0, pl.ds(row, 1), :], 0.0),
                         axis=(0, 1), keepdims=True)
            ny = jnp.sum(jnp.where(lsel, cxyz_ref[b, 1, pl.ds(row, 1), :], 0.0),
                         axis=(0, 1), keepdims=True)
            nz = jnp.sum(jnp.where(lsel, cxyz_ref[b, 2, pl.ds(row, 1), :], 0.0),
                         axis=(0, 1), keepdims=True)
            at_i = iota_m == i
            sel = jnp.where(at_i, idx, sel)
            selx = jnp.where(at_i, nx, selx)
            sely = jnp.where(at_i, ny, sely)
            selz = jnp.where(at_i, nz, selz)
            new.append((dist, nx, ny, nz, sel, selx, sely, selz))
        return tuple(new)

    final = jax.lax.fori_loop(1, num_sel, body, tuple(inits))
    for b in range(B):
        _, _, _, _, sel, selx, sely, selz = final[b]
        sel_ref[b] = sel
        sxyz_ref[b, 0] = selx
        sxyz_ref[b, 1] = sely
        sxyz_ref[b, 2] = selz


_KNN_R = 128
_CHUNK = 128
_NCAND = 36


def _knn_p1_body(c_ref, sxyz_ref, d_ref, cid_ref, gidx_ref):
    N = c_ref.shape[2]
    R = sxyz_ref.shape[2]
    T = cid_ref.shape[2]
    nchunk = N // _CHUNK
    comps = []
    for c in range(3):
        cc = jnp.reshape(c_ref[0, c, :], (1, N))
        sc = jnp.reshape(sxyz_ref[0, c, :], (R, 1))
        comps.append((sc, cc))
    dx = comps[0][0] - comps[0][1]
    dy = comps[1][0] - comps[1][1]
    dz = comps[2][0] - comps[2][1]
    d = (dx * dx + dy * dy) + dz * dz
    d_ref[0] = d
    cmin = jnp.min(d.reshape(R, nchunk, _CHUNK), axis=2)
    cgidx = jax.lax.broadcasted_iota(jnp.int32, (R, nchunk), 1)
    lane_c = jax.lax.broadcasted_iota(jnp.int32, (1, _CHUNK), 1)
    for t in range(T):
        m = jnp.min(cmin, axis=1, keepdims=True)
        cid = jnp.min(jnp.where(cmin == m, cgidx, _BIG), axis=1, keepdims=True)
        cid_ref[0, :, t : t + 1] = cid
        gidx_ref[0, :, t * _CHUNK : (t + 1) * _CHUNK] = cid * _CHUNK + lane_c
        cmin = jnp.where(cgidx == cid, jnp.inf, cmin)


def _knn_p2_body(cand_ref, gidx_ref, nidx_ref, vals_ref):
    G = nidx_ref.shape[2]
    vals_ref[...] = cand_ref[0]
    g = gidx_ref[0]
    for k in range(G):
        vals = vals_ref[...]
        m = jnp.min(vals, axis=1, keepdims=True)
        idx = jnp.min(jnp.where(vals == m, g, _BIG), axis=1, keepdims=True)
        nidx_ref[0, :, k : k + 1] = idx
        vals_ref[...] = jnp.where(g == idx, jnp.inf, vals)


def _sc_compact(table, idx):
    from jax.experimental.pallas import tpu_sc as plsc

    NI = idx.shape[0]
    NW = 32
    NSUB = 4
    b_per_w = NI // NW
    b_sub = b_per_w // NSUB
    D = table.shape[1]
    mesh = plsc.VectorSubcoreMesh(core_axis_name="c", subcore_axis_name="s")

    @functools.partial(
        pl.kernel,
        mesh=mesh,
        out_type=jax.ShapeDtypeStruct((NI, D), jnp.float32),
        scratch_types=[
            pltpu.VMEM((b_sub,), jnp.int32),
            pltpu.VMEM((b_sub, D), jnp.float32),
            pltpu.SemaphoreType.DMA,
        ],
    )
    def gath(table_hbm, idx_hbm, out_hbm, idx_v, rows_v, sem):
        wid = jax.lax.axis_index("s") * 2 + jax.lax.axis_index("c")
        for g in range(NSUB):
            base = wid * b_per_w + g * b_sub
            pltpu.sync_copy(idx_hbm.at[pl.ds(base, b_sub)], idx_v)
            pltpu.async_copy(table_hbm.at[idx_v], rows_v, sem).wait()
            pltpu.sync_copy(rows_v, out_hbm.at[pl.ds(base, b_sub)])

    return gath(table, idx)


def _knn_pallas(c_t, sxyz_t, B, n_per, num_group, group_size):
    N, M, T = n_per, num_group, _NCAND
    nblk = M // _KNN_R
    nchunk = N // _CHUNK
    d_out, chunkids, gidx = pl.pallas_call(
        _knn_p1_body,
        grid=(B, nblk),
        in_specs=[
            pl.BlockSpec((1, 3, N), lambda b, rb: (b, 0, 0)),
            pl.BlockSpec((1, 3, _KNN_R), lambda b, rb: (b, 0, rb)),
        ],
        out_specs=[
            pl.BlockSpec((1, _KNN_R, N), lambda b, rb: (b, rb, 0)),
            pl.BlockSpec((1, _KNN_R, T), lambda b, rb: (b, rb, 0)),
            pl.BlockSpec((1, _KNN_R, T * _CHUNK), lambda b, rb: (b, rb, 0)),
        ],
        out_shape=[
            jax.ShapeDtypeStruct((B, M, N), jnp.float32),
            jax.ShapeDtypeStruct((B, M, T), jnp.int32),
            jax.ShapeDtypeStruct((B, M, T * _CHUNK), jnp.int32),
        ],
    )(c_t, sxyz_t)

    row_base = (jnp.arange(B, dtype=jnp.int32)[:, None, None] * M
                + jnp.arange(M, dtype=jnp.int32)[None, :, None]) * nchunk
    idx_flat = (row_base + chunkids).reshape(-1)
    table = d_out.reshape(B * M * nchunk, _CHUNK)
    compact = _sc_compact(table, idx_flat).reshape(B, M, T * _CHUNK)

    return pl.pallas_call(
        _knn_p2_body,
        grid=(B,),
        in_specs=[
            pl.BlockSpec((1, M, T * _CHUNK), lambda b: (b, 0, 0)),
            pl.BlockSpec((1, M, T * _CHUNK), lambda b: (b, 0, 0)),
        ],
        out_specs=pl.BlockSpec((1, M, group_size), lambda b: (b, 0, 0)),
        out_shape=jax.ShapeDtypeStruct((B, M, group_size), jnp.int32),
        scratch_shapes=[pltpu.VMEM((M, T * _CHUNK), jnp.float32)],
    )(compact, gidx)


def _fps_pallas(coord, B, n_per, num_group):
    m_rows = num_group // _LANE
    c_t = coord.reshape(B, n_per, 3).transpose(0, 2, 1).reshape(B, 3, _SUB, _LANE)
    sel8, sxyz8 = pl.pallas_call(
        _fps_body,
        out_shape=[
            jax.ShapeDtypeStruct((B, m_rows, _LANE), jnp.int32),
            jax.ShapeDtypeStruct((B, 3, m_rows, _LANE), jnp.float32),
        ],
    )(c_t)
    local_idx = sel8.reshape(B, num_group)
    s_xyz_b = sxyz8.reshape(B, 3, num_group).transpose(0, 2, 1)
    return local_idx, s_xyz_b


def kernel(coord, offset, batch_bin, serialized_code):
    B = offset.shape[0]
    n_total = coord.shape[0]
    n_per = n_total // B
    num_group = 512
    group_size = 32

    local_idx, s_xyz_b = _fps_pallas(coord, B, n_per, num_group)
    base = (jnp.arange(B, dtype=jnp.int32) * n_per)[:, None]
    s_idx = (local_idx + base).reshape(-1)

    c_t = coord.reshape(B, n_per, 3).transpose(0, 2, 1)
    sxyz_t = s_xyz_b.transpose(0, 2, 1)
    nidx_local = _knn_pallas(c_t, sxyz_t, B, n_per, num_group, group_size)
    nidx = (nidx_local + base[:, :, None]).reshape(B * num_group, group_size)

    s_xyz = s_xyz_b.reshape(B * num_group, 3)
    s_n = coord[nidx] - s_xyz[:, None, :]
    s_n = s_n[:, 1:, :]
    codes = serialized_code[:, s_idx]
    s_order, s_inverse = _argsort_inverse(codes)
    return (s_idx, s_n, s_xyz, s_order, s_inverse)
